# baseline (speedup 1.0000x reference)
import functools

import jax
import jax.numpy as jnp
from jax import lax
from jax.experimental import pallas as pl
from jax.experimental.pallas import tpu as pltpu
from jax.experimental.pallas import tpu_sc as plsc

NC, NS = 2, 16
NW = NC * NS
C = 128
R = 1000


def _cdiv(a, b):
    return (a + b - 1) // b


@functools.lru_cache(maxsize=None)
def _make_agg(width, ept, acc_r):
    cpt = ept // C
    rpt = acc_r // NS
    mesh = plsc.VectorSubcoreMesh(core_axis_name="c", subcore_axis_name="s",
                                  num_cores=NC, num_subcores=NS)

    @functools.partial(
        pl.kernel,
        out_type=jax.ShapeDtypeStruct((NC, acc_r, width), jnp.float32),
        mesh=mesh,
        scratch_types=[
            pltpu.VMEM((8, C), jnp.int32),
            pltpu.VMEM((C, width), jnp.float32),
            pltpu.VMEM_SHARED((acc_r, width), jnp.float32),
            pltpu.SemaphoreType.DMA,
        ],
    )
    def agg(table, sd3d, zeros, out, sd_v, rows_v, acc, sem):
        cid = lax.axis_index("c")
        sid = lax.axis_index("s")
        w = cid * NS + sid
        pltpu.sync_copy(zeros, rows_v)
        base_r = sid * rpt
        for k in range(rpt // C):
            pltpu.sync_copy(rows_v, acc.at[pl.ds(base_r + k * C, C)])
        plsc.subcore_barrier()

        base_c = w * cpt

        def chunk(j, carry):
            pltpu.sync_copy(sd3d.at[base_c + j], sd_v)
            pltpu.async_copy(table.at[sd_v.at[0]], rows_v, sem).wait()
            pltpu.sync_copy(rows_v, acc.at[sd_v.at[1]], add=True)
            return carry

        lax.fori_loop(0, cpt, chunk, 0)
        plsc.subcore_barrier()
        for k in range(rpt // C):
            r0 = base_r + k * C
            pltpu.sync_copy(acc.at[pl.ds(r0, C)], out.at[cid, pl.ds(r0, C)])

    return agg


@functools.lru_cache(maxsize=None)
def _make_deg(width, ept, acc_r):
    cpt = ept // C
    rpt = acc_r // NS
    mesh = plsc.VectorSubcoreMesh(core_axis_name="c", subcore_axis_name="s",
                                  num_cores=NC, num_subcores=NS)

    GB = 8

    @functools.partial(
        pl.kernel,
        out_type=jax.ShapeDtypeStruct((NC, acc_r, width), jnp.float32),
        mesh=mesh,
        scratch_types=[
            pltpu.VMEM((cpt, C), jnp.int32),
            pltpu.VMEM((C, width), jnp.float32),
            pltpu.VMEM_SHARED((acc_r, width), jnp.float32),
            pltpu.SemaphoreType.DMA,
        ],
    )
    def deg(dst2d, zeros, ones, out, dsts, rows_v, acc, sem):
        cid = lax.axis_index("c")
        sid = lax.axis_index("s")
        w = cid * NS + sid
        pltpu.sync_copy(dst2d.at[pl.ds(w * cpt, cpt)], dsts)
        pltpu.sync_copy(zeros, rows_v)
        base_r = sid * rpt
        for k in range(rpt // C):
            pltpu.sync_copy(rows_v, acc.at[pl.ds(base_r + k * C, C)])
        plsc.subcore_barrier()
        pltpu.sync_copy(ones, rows_v)

        def grp(g, carry):
            for b in range(GB):
                pltpu.async_copy(rows_v, acc.at[dsts.at[g * GB + b]], sem,
                                 add=True)
            for b in range(GB):
                pltpu.make_async_copy(rows_v, acc.at[dsts.at[0]], sem).wait()
            return carry

        lax.fori_loop(0, cpt // GB, grp, 0)
        plsc.subcore_barrier()
        for k in range(rpt // C):
            r0 = base_r + k * C
            pltpu.sync_copy(acc.at[pl.ds(r0, C)], out.at[cid, pl.ds(r0, C)])

    return deg


def _tc0(x, w0, degp, n, f):
    g = n // R

    def body(x_ref, w_ref, d0_ref, d1_ref, g_ref, dinv_ref):
        deg = d0_ref[...][0, :, 0:1] + d1_ref[...][0, :, 0:1] + 1.0
        dinv = lax.rsqrt(deg)
        hw = lax.dot_general(x_ref[...], w_ref[...], (((1,), (1,)), ((), ())),
                             preferred_element_type=jnp.float32)
        g_ref[...] = dinv * hw
        dinv_ref[...] = dinv

    return pl.pallas_call(
        body,
        grid=(g,),
        in_specs=[
            pl.BlockSpec((R, f), lambda i: (i, 0)),
            pl.BlockSpec((f, f), lambda i: (0, 0)),
            pl.BlockSpec((1, R, f), lambda i: (0, i, 0)),
            pl.BlockSpec((1, R, f), lambda i: (1, i, 0)),
        ],
        out_specs=[
            pl.BlockSpec((R, f), lambda i: (i, 0)),
            pl.BlockSpec((R, 1), lambda i: (i, 0)),
        ],
        out_shape=[
            jax.ShapeDtypeStruct((n, f), jnp.float32),
            jax.ShapeDtypeStruct((n, 1), jnp.float32),
        ],
    )(x, w0, degp, degp)


def _tc_mid(p, g_prev, dinv, b_prev, w_next, n, f):
    g = n // R

    def body(p0_ref, p1_ref, g_ref, dinv_ref, b_ref, w_ref, out_ref):
        agg = p0_ref[...][0] + p1_ref[...][0] + g_ref[...]
        h = jax.nn.sigmoid(dinv_ref[...] * agg + b_ref[...])
        hw = lax.dot_general(h, w_ref[...], (((1,), (1,)), ((), ())),
                             preferred_element_type=jnp.float32)
        out_ref[...] = dinv_ref[...] * hw

    return pl.pallas_call(
        body,
        grid=(g,),
        in_specs=[
            pl.BlockSpec((1, R, f), lambda i: (0, i, 0)),
            pl.BlockSpec((1, R, f), lambda i: (1, i, 0)),
            pl.BlockSpec((R, f), lambda i: (i, 0)),
            pl.BlockSpec((R, 1), lambda i: (i, 0)),
            pl.BlockSpec((1, f), lambda i: (0, 0)),
            pl.BlockSpec((f, f), lambda i: (0, 0)),
        ],
        out_shape=jax.ShapeDtypeStruct((n, f), jnp.float32),
        out_specs=pl.BlockSpec((R, f), lambda i: (i, 0)),
    )(p, p, g_prev, dinv, b_prev.reshape(1, f), w_next)


def _tc2(p, g_prev, dinv, b_prev, w2, n, f):
    g = n // R

    def body(p0_ref, p1_ref, g_ref, dinv_ref, b_ref, w_ref, out_ref):
        agg = p0_ref[...][0] + p1_ref[...][0] + g_ref[...]
        h = jax.nn.sigmoid(dinv_ref[...] * agg + b_ref[...])
        s = lax.dot_general(h, w_ref[...], (((1,), (1,)), ((), ())),
                            preferred_element_type=jnp.float32)
        out_ref[...] = jnp.broadcast_to(dinv_ref[...] * s, (R, f))

    return pl.pallas_call(
        body,
        grid=(g,),
        in_specs=[
            pl.BlockSpec((1, R, f), lambda i: (0, i, 0)),
            pl.BlockSpec((1, R, f), lambda i: (1, i, 0)),
            pl.BlockSpec((R, f), lambda i: (i, 0)),
            pl.BlockSpec((R, 1), lambda i: (i, 0)),
            pl.BlockSpec((1, f), lambda i: (0, 0)),
            pl.BlockSpec((1, f), lambda i: (0, 0)),
        ],
        out_shape=jax.ShapeDtypeStruct((n, f), jnp.float32),
        out_specs=pl.BlockSpec((R, f), lambda i: (i, 0)),
    )(p, p, g_prev, dinv, b_prev.reshape(1, f), w2)


def _tc3(p, g2, dinv, b2, n, f):
    g = n // R

    def body(p0_ref, p1_ref, g_ref, dinv_ref, b_ref, out_ref):
        agg = (p0_ref[...][0, :, 0:1] + p1_ref[...][0, :, 0:1]
               + g_ref[...][:, 0:1])
        out_ref[...] = jax.nn.sigmoid(dinv_ref[...] * agg + b_ref[...]) + 1e-6

    return pl.pallas_call(
        body,
        grid=(g,),
        in_specs=[
            pl.BlockSpec((1, R, f), lambda i: (0, i, 0)),
            pl.BlockSpec((1, R, f), lambda i: (1, i, 0)),
            pl.BlockSpec((R, f), lambda i: (i, 0)),
            pl.BlockSpec((R, 1), lambda i: (i, 0)),
            pl.BlockSpec((1, 1), lambda i: (0, 0)),
        ],
        out_shape=jax.ShapeDtypeStruct((n, 1), jnp.float32),
        out_specs=pl.BlockSpec((R, 1), lambda i: (i, 0)),
    )(p, p, g2, dinv, b2.reshape(1, 1))


def kernel(x, edge_index, batch, W0, b0, W1, b1, W2, b2):
    n, f = x.shape
    e = edge_index.shape[1]
    epad = _cdiv(e, NW * C * 8) * NW * C * 8
    ept = epad // NW
    acc_r = _cdiv(n + 1, NS * C) * NS * C

    src = jnp.concatenate(
        [edge_index[0].astype(jnp.int32), jnp.zeros((epad - e,), jnp.int32)])
    dst = jnp.concatenate(
        [edge_index[1].astype(jnp.int32), jnp.full((epad - e,), n, jnp.int32)])
    nc = epad // C
    dst2d = dst.reshape(nc, C)
    sd3d = jnp.concatenate(
        [src.reshape(nc, 1, C), dst.reshape(nc, 1, C),
         jnp.zeros((nc, 6, C), jnp.int32)], axis=1)
    zeros_f = jnp.zeros((C, f), jnp.float32)
    ones_f = jnp.ones((C, f), jnp.float32)

    agg_f = _make_agg(f, ept, acc_r)
    deg_f = _make_deg(f, ept, acc_r)

    degp = deg_f(dst2d, zeros_f, ones_f)
    g0, dinv = _tc0(x, W0, degp, n, f)
    p0 = agg_f(g0, sd3d, zeros_f)
    g1 = _tc_mid(p0, g0, dinv, b0, W1, n, f)
    p1 = agg_f(g1, sd3d, zeros_f)
    g2 = _tc2(p1, g1, dinv, b1, W2, n, f)
    p2 = agg_f(g2, sd3d, zeros_f)
    return _tc3(p2, g2, dinv, b2, n, f)


# speedup vs baseline: 1.5780x; 1.5780x over previous
import functools

import jax
import jax.numpy as jnp
from jax import lax
from jax.experimental import pallas as pl
from jax.experimental.pallas import tpu as pltpu
from jax.experimental.pallas import tpu_sc as plsc

NC, NS = 2, 16
NW = NC * NS
C = 128
R = 1000


def _cdiv(a, b):
    return (a + b - 1) // b


@functools.lru_cache(maxsize=None)
def _make_agg(width, ept, acc_r):
    cpt = ept // C
    rpt = acc_r // NS
    mesh = plsc.VectorSubcoreMesh(core_axis_name="c", subcore_axis_name="s",
                                  num_cores=NC, num_subcores=NS)

    @functools.partial(
        pl.kernel,
        out_type=jax.ShapeDtypeStruct((NC, acc_r, width), jnp.float32),
        mesh=mesh,
        scratch_types=[
            pltpu.VMEM((C,), jnp.int32),
            pltpu.VMEM((C,), jnp.int32),
            pltpu.VMEM((C, width), jnp.float32),
            pltpu.VMEM_SHARED((acc_r, width), jnp.float32),
            pltpu.SemaphoreType.DMA,
        ],
    )
    def agg(table, src1d, dst1d, zeros, out, src_v, dst_v, rows_v, acc, sem):
        cid = lax.axis_index("c")
        sid = lax.axis_index("s")
        w = cid * NS + sid
        pltpu.sync_copy(zeros, rows_v)
        base_r = sid * rpt
        for k in range(rpt // C):
            pltpu.sync_copy(rows_v, acc.at[pl.ds(base_r + k * C, C)])
        plsc.subcore_barrier()

        base_e = w * ept

        def chunk(j, carry):
            off = base_e + j * C
            pltpu.sync_copy(src1d.at[pl.ds(off, C)], src_v)
            pltpu.sync_copy(dst1d.at[pl.ds(off, C)], dst_v)
            pltpu.async_copy(table.at[src_v], rows_v, sem).wait()
            pltpu.sync_copy(rows_v, acc.at[dst_v], add=True)
            return carry

        lax.fori_loop(0, cpt, chunk, 0)
        plsc.subcore_barrier()
        for k in range(rpt // C):
            r0 = base_r + k * C
            pltpu.sync_copy(acc.at[pl.ds(r0, C)], out.at[cid, pl.ds(r0, C)])

    return agg


@functools.lru_cache(maxsize=None)
def _make_deg(width, ept, acc_r):
    cpt = ept // C
    rpt = acc_r // NS
    mesh = plsc.VectorSubcoreMesh(core_axis_name="c", subcore_axis_name="s",
                                  num_cores=NC, num_subcores=NS)

    GB = 8

    @functools.partial(
        pl.kernel,
        out_type=jax.ShapeDtypeStruct((NC, acc_r, width), jnp.float32),
        mesh=mesh,
        scratch_types=[
            [pltpu.VMEM((C,), jnp.int32)] * GB,
            pltpu.VMEM((C, width), jnp.float32),
            pltpu.VMEM_SHARED((acc_r, width), jnp.float32),
            pltpu.SemaphoreType.DMA,
        ],
    )
    def deg(dst1d, zeros, ones, out, dstb, rows_v, acc, sem):
        cid = lax.axis_index("c")
        sid = lax.axis_index("s")
        w = cid * NS + sid
        pltpu.sync_copy(zeros, rows_v)
        base_r = sid * rpt
        for k in range(rpt // C):
            pltpu.sync_copy(rows_v, acc.at[pl.ds(base_r + k * C, C)])
        plsc.subcore_barrier()
        pltpu.sync_copy(ones, rows_v)

        base_e = w * ept
        ngrp = cpt // GB

        def grp(g, carry):
            for b in range(GB):
                off = base_e + (g * GB + b) * C
                pltpu.sync_copy(dst1d.at[pl.ds(off, C)], dstb[b])
                pltpu.async_copy(rows_v, acc.at[dstb[b]], sem, add=True)
            for b in range(GB):
                pltpu.make_async_copy(rows_v, acc.at[dstb[0]], sem).wait()
            return carry

        lax.fori_loop(0, ngrp, grp, 0)
        for t in range(cpt - ngrp * GB):
            off = base_e + (ngrp * GB + t) * C
            pltpu.sync_copy(dst1d.at[pl.ds(off, C)], dstb[0])
            pltpu.sync_copy(rows_v, acc.at[dstb[0]], add=True)
        plsc.subcore_barrier()
        for k in range(rpt // C):
            r0 = base_r + k * C
            pltpu.sync_copy(acc.at[pl.ds(r0, C)], out.at[cid, pl.ds(r0, C)])

    return deg


def _tc0(x, w0, degp, n, f):
    g = n // R

    def body(x_ref, w_ref, d0_ref, d1_ref, g_ref, dinv_ref):
        deg = d0_ref[...][0, :, 0:1] + d1_ref[...][0, :, 0:1] + 1.0
        dinv = lax.rsqrt(deg)
        hw = lax.dot_general(x_ref[...], w_ref[...], (((1,), (1,)), ((), ())),
                             preferred_element_type=jnp.float32)
        g_ref[...] = dinv * hw
        dinv_ref[...] = dinv

    return pl.pallas_call(
        body,
        grid=(g,),
        in_specs=[
            pl.BlockSpec((R, f), lambda i: (i, 0)),
            pl.BlockSpec((f, f), lambda i: (0, 0)),
            pl.BlockSpec((1, R, f), lambda i: (0, i, 0)),
            pl.BlockSpec((1, R, f), lambda i: (1, i, 0)),
        ],
        out_specs=[
            pl.BlockSpec((R, f), lambda i: (i, 0)),
            pl.BlockSpec((R, 1), lambda i: (i, 0)),
        ],
        out_shape=[
            jax.ShapeDtypeStruct((n, f), jnp.float32),
            jax.ShapeDtypeStruct((n, 1), jnp.float32),
        ],
    )(x, w0, degp, degp)


def _tc_mid(p, g_prev, dinv, b_prev, w_next, n, f):
    g = n // R

    def body(p0_ref, p1_ref, g_ref, dinv_ref, b_ref, w_ref, out_ref):
        agg = p0_ref[...][0] + p1_ref[...][0] + g_ref[...]
        h = jax.nn.sigmoid(dinv_ref[...] * agg + b_ref[...])
        hw = lax.dot_general(h, w_ref[...], (((1,), (1,)), ((), ())),
                             preferred_element_type=jnp.float32)
        out_ref[...] = dinv_ref[...] * hw

    return pl.pallas_call(
        body,
        grid=(g,),
        in_specs=[
            pl.BlockSpec((1, R, f), lambda i: (0, i, 0)),
            pl.BlockSpec((1, R, f), lambda i: (1, i, 0)),
            pl.BlockSpec((R, f), lambda i: (i, 0)),
            pl.BlockSpec((R, 1), lambda i: (i, 0)),
            pl.BlockSpec((1, f), lambda i: (0, 0)),
            pl.BlockSpec((f, f), lambda i: (0, 0)),
        ],
        out_shape=jax.ShapeDtypeStruct((n, f), jnp.float32),
        out_specs=pl.BlockSpec((R, f), lambda i: (i, 0)),
    )(p, p, g_prev, dinv, b_prev.reshape(1, f), w_next)


def _tc2(p, g_prev, dinv, b_prev, w2, n, f):
    g = n // R

    def body(p0_ref, p1_ref, g_ref, dinv_ref, b_ref, w_ref, out_ref):
        agg = p0_ref[...][0] + p1_ref[...][0] + g_ref[...]
        h = jax.nn.sigmoid(dinv_ref[...] * agg + b_ref[...])
        s = lax.dot_general(h, w_ref[...], (((1,), (1,)), ((), ())),
                            preferred_element_type=jnp.float32)
        out_ref[...] = jnp.broadcast_to(dinv_ref[...] * s, (R, f))

    return pl.pallas_call(
        body,
        grid=(g,),
        in_specs=[
            pl.BlockSpec((1, R, f), lambda i: (0, i, 0)),
            pl.BlockSpec((1, R, f), lambda i: (1, i, 0)),
            pl.BlockSpec((R, f), lambda i: (i, 0)),
            pl.BlockSpec((R, 1), lambda i: (i, 0)),
            pl.BlockSpec((1, f), lambda i: (0, 0)),
            pl.BlockSpec((1, f), lambda i: (0, 0)),
        ],
        out_shape=jax.ShapeDtypeStruct((n, f), jnp.float32),
        out_specs=pl.BlockSpec((R, f), lambda i: (i, 0)),
    )(p, p, g_prev, dinv, b_prev.reshape(1, f), w2)


def _tc3(p, g2, dinv, b2, n, f):
    g = n // R

    def body(p0_ref, p1_ref, g_ref, dinv_ref, b_ref, out_ref):
        agg = (p0_ref[...][0, :, 0:1] + p1_ref[...][0, :, 0:1]
               + g_ref[...][:, 0:1])
        out_ref[...] = jax.nn.sigmoid(dinv_ref[...] * agg + b_ref[...]) + 1e-6

    return pl.pallas_call(
        body,
        grid=(g,),
        in_specs=[
            pl.BlockSpec((1, R, f), lambda i: (0, i, 0)),
            pl.BlockSpec((1, R, f), lambda i: (1, i, 0)),
            pl.BlockSpec((R, f), lambda i: (i, 0)),
            pl.BlockSpec((R, 1), lambda i: (i, 0)),
            pl.BlockSpec((1, 1), lambda i: (0, 0)),
        ],
        out_shape=jax.ShapeDtypeStruct((n, 1), jnp.float32),
        out_specs=pl.BlockSpec((R, 1), lambda i: (i, 0)),
    )(p, p, g2, dinv, b2.reshape(1, 1))


def kernel(x, edge_index, batch, W0, b0, W1, b1, W2, b2):
    n, f = x.shape
    e = edge_index.shape[1]
    epad = _cdiv(e, NW * C) * NW * C
    ept = epad // NW
    acc_r = _cdiv(n + 1, NS * C) * NS * C

    src = jnp.concatenate(
        [edge_index[0].astype(jnp.int32), jnp.zeros((epad - e,), jnp.int32)])
    dst = jnp.concatenate(
        [edge_index[1].astype(jnp.int32), jnp.full((epad - e,), n, jnp.int32)])
    zeros_f = jnp.zeros((C, f), jnp.float32)
    ones_f = jnp.ones((C, f), jnp.float32)

    agg_f = _make_agg(f, ept, acc_r)
    deg_f = _make_deg(f, ept, acc_r)

    degp = deg_f(dst, zeros_f, ones_f)
    g0, dinv = _tc0(x, W0, degp, n, f)
    p0 = agg_f(g0, src, dst, zeros_f)
    g1 = _tc_mid(p0, g0, dinv, b0, W1, n, f)
    p1 = agg_f(g1, src, dst, zeros_f)
    g2 = _tc2(p1, g1, dinv, b1, W2, n, f)
    p2 = agg_f(g2, src, dst, zeros_f)
    return _tc3(p2, g2, dinv, b2, n, f)


# speedup vs baseline: 1.7721x; 1.1230x over previous
import functools

import jax
import jax.numpy as jnp
from jax import lax
from jax.experimental import pallas as pl
from jax.experimental.pallas import tpu as pltpu
from jax.experimental.pallas import tpu_sc as plsc

NC, NS = 2, 16
NW = NC * NS
C = 128
R = 1000


def _cdiv(a, b):
    return (a + b - 1) // b


@functools.lru_cache(maxsize=None)
def _make_agg(width, ept, acc_r):
    cpt = ept // C
    rpt = acc_r // NS
    mesh = plsc.VectorSubcoreMesh(core_axis_name="c", subcore_axis_name="s",
                                  num_cores=NC, num_subcores=NS)

    @functools.partial(
        pl.kernel,
        out_type=jax.ShapeDtypeStruct((NC, acc_r, width), jnp.float32),
        mesh=mesh,
        scratch_types=[
            pltpu.VMEM((C,), jnp.int32),
            pltpu.VMEM((C,), jnp.int32),
            pltpu.VMEM((C,), jnp.int32),
            pltpu.VMEM((C, width), jnp.float32),
            pltpu.VMEM((C, width), jnp.float32),
            pltpu.VMEM_SHARED((acc_r, width), jnp.float32),
            pltpu.SemaphoreType.DMA,
            pltpu.SemaphoreType.DMA,
            pltpu.SemaphoreType.DMA,
        ],
    )
    def agg(table, src1d, dst1d, zeros, out,
            src_v, dstb0, dstb1, rows0, rows1, acc, semg, sems0, sems1):
        cid = lax.axis_index("c")
        sid = lax.axis_index("s")
        w = cid * NS + sid
        pltpu.sync_copy(zeros, rows0)
        base_r = sid * rpt
        for k in range(rpt // C):
            pltpu.sync_copy(rows0, acc.at[pl.ds(base_r + k * C, C)])
        plsc.subcore_barrier()

        base_e = w * ept

        def body(jj, carry):
            c0 = base_e + 2 * jj * C
            c1 = c0 + C
            @pl.when(jj > 0)
            def _():
                pltpu.make_async_copy(rows0, acc.at[dstb0], sems0).wait()
            pltpu.sync_copy(src1d.at[pl.ds(c0, C)], src_v)
            pltpu.sync_copy(dst1d.at[pl.ds(c0, C)], dstb0)
            pltpu.async_copy(table.at[src_v], rows0, semg).wait()
            pltpu.async_copy(rows0, acc.at[dstb0], sems0, add=True)

            @pl.when(jj > 0)
            def _():
                pltpu.make_async_copy(rows1, acc.at[dstb1], sems1).wait()
            pltpu.sync_copy(src1d.at[pl.ds(c1, C)], src_v)
            pltpu.sync_copy(dst1d.at[pl.ds(c1, C)], dstb1)
            pltpu.async_copy(table.at[src_v], rows1, semg).wait()
            pltpu.async_copy(rows1, acc.at[dstb1], sems1, add=True)
            return carry

        lax.fori_loop(0, cpt // 2, body, 0)
        pltpu.make_async_copy(rows0, acc.at[dstb0], sems0).wait()
        pltpu.make_async_copy(rows1, acc.at[dstb1], sems1).wait()
        for t in range(cpt - (cpt // 2) * 2):
            off = base_e + (cpt - 1) * C
            pltpu.sync_copy(src1d.at[pl.ds(off, C)], src_v)
            pltpu.sync_copy(dst1d.at[pl.ds(off, C)], dstb0)
            pltpu.async_copy(table.at[src_v], rows0, semg).wait()
            pltpu.sync_copy(rows0, acc.at[dstb0], add=True)
        plsc.subcore_barrier()
        for k in range(rpt // C):
            r0 = base_r + k * C
            pltpu.sync_copy(acc.at[pl.ds(r0, C)], out.at[cid, pl.ds(r0, C)])

    return agg


@functools.lru_cache(maxsize=None)
def _make_deg(width, ept, acc_r):
    cpt = ept // C
    rpt = acc_r // NS
    mesh = plsc.VectorSubcoreMesh(core_axis_name="c", subcore_axis_name="s",
                                  num_cores=NC, num_subcores=NS)

    GB = 8

    @functools.partial(
        pl.kernel,
        out_type=jax.ShapeDtypeStruct((NC, acc_r, width), jnp.float32),
        mesh=mesh,
        scratch_types=[
            [pltpu.VMEM((C,), jnp.int32)] * GB,
            pltpu.VMEM((C, width), jnp.float32),
            pltpu.VMEM_SHARED((acc_r, width), jnp.float32),
            pltpu.SemaphoreType.DMA,
        ],
    )
    def deg(dst1d, zeros, ones, out, dstb, rows_v, acc, sem):
        cid = lax.axis_index("c")
        sid = lax.axis_index("s")
        w = cid * NS + sid
        pltpu.sync_copy(zeros, rows_v)
        base_r = sid * rpt
        for k in range(rpt // C):
            pltpu.sync_copy(rows_v, acc.at[pl.ds(base_r + k * C, C)])
        plsc.subcore_barrier()
        pltpu.sync_copy(ones, rows_v)

        base_e = w * ept
        ngrp = cpt // GB

        def grp(g, carry):
            for b in range(GB):
                off = base_e + (g * GB + b) * C
                pltpu.sync_copy(dst1d.at[pl.ds(off, C)], dstb[b])
                pltpu.async_copy(rows_v, acc.at[dstb[b]], sem, add=True)
            for b in range(GB):
                pltpu.make_async_copy(rows_v, acc.at[dstb[0]], sem).wait()
            return carry

        lax.fori_loop(0, ngrp, grp, 0)
        for t in range(cpt - ngrp * GB):
            off = base_e + (ngrp * GB + t) * C
            pltpu.sync_copy(dst1d.at[pl.ds(off, C)], dstb[0])
            pltpu.sync_copy(rows_v, acc.at[dstb[0]], add=True)
        plsc.subcore_barrier()
        for k in range(rpt // C):
            r0 = base_r + k * C
            pltpu.sync_copy(acc.at[pl.ds(r0, C)], out.at[cid, pl.ds(r0, C)])

    return deg


def _tc0(x, w0, degp, n, f):
    g = n // R

    def body(x_ref, w_ref, d0_ref, d1_ref, g_ref, dinv_ref):
        deg = d0_ref[...][0, :, 0:1] + d1_ref[...][0, :, 0:1] + 1.0
        dinv = lax.rsqrt(deg)
        hw = lax.dot_general(x_ref[...], w_ref[...], (((1,), (1,)), ((), ())),
                             preferred_element_type=jnp.float32)
        g_ref[...] = dinv * hw
        dinv_ref[...] = dinv

    return pl.pallas_call(
        body,
        grid=(g,),
        in_specs=[
            pl.BlockSpec((R, f), lambda i: (i, 0)),
            pl.BlockSpec((f, f), lambda i: (0, 0)),
            pl.BlockSpec((1, R, f), lambda i: (0, i, 0)),
            pl.BlockSpec((1, R, f), lambda i: (1, i, 0)),
        ],
        out_specs=[
            pl.BlockSpec((R, f), lambda i: (i, 0)),
            pl.BlockSpec((R, 1), lambda i: (i, 0)),
        ],
        out_shape=[
            jax.ShapeDtypeStruct((n, f), jnp.float32),
            jax.ShapeDtypeStruct((n, 1), jnp.float32),
        ],
    )(x, w0, degp, degp)


def _tc_mid(p, g_prev, dinv, b_prev, w_next, n, f):
    g = n // R

    def body(p0_ref, p1_ref, g_ref, dinv_ref, b_ref, w_ref, out_ref):
        agg = p0_ref[...][0] + p1_ref[...][0] + g_ref[...]
        h = jax.nn.sigmoid(dinv_ref[...] * agg + b_ref[...])
        hw = lax.dot_general(h, w_ref[...], (((1,), (1,)), ((), ())),
                             preferred_element_type=jnp.float32)
        out_ref[...] = dinv_ref[...] * hw

    return pl.pallas_call(
        body,
        grid=(g,),
        in_specs=[
            pl.BlockSpec((1, R, f), lambda i: (0, i, 0)),
            pl.BlockSpec((1, R, f), lambda i: (1, i, 0)),
            pl.BlockSpec((R, f), lambda i: (i, 0)),
            pl.BlockSpec((R, 1), lambda i: (i, 0)),
            pl.BlockSpec((1, f), lambda i: (0, 0)),
            pl.BlockSpec((f, f), lambda i: (0, 0)),
        ],
        out_shape=jax.ShapeDtypeStruct((n, f), jnp.float32),
        out_specs=pl.BlockSpec((R, f), lambda i: (i, 0)),
    )(p, p, g_prev, dinv, b_prev.reshape(1, f), w_next)


def _tc2(p, g_prev, dinv, b_prev, w2, n, f):
    g = n // R

    def body(p0_ref, p1_ref, g_ref, dinv_ref, b_ref, w_ref, out_ref):
        agg = p0_ref[...][0] + p1_ref[...][0] + g_ref[...]
        h = jax.nn.sigmoid(dinv_ref[...] * agg + b_ref[...])
        s = lax.dot_general(h, w_ref[...], (((1,), (1,)), ((), ())),
                            preferred_element_type=jnp.float32)
        out_ref[...] = jnp.broadcast_to(dinv_ref[...] * s, (R, f))

    return pl.pallas_call(
        body,
        grid=(g,),
        in_specs=[
            pl.BlockSpec((1, R, f), lambda i: (0, i, 0)),
            pl.BlockSpec((1, R, f), lambda i: (1, i, 0)),
            pl.BlockSpec((R, f), lambda i: (i, 0)),
            pl.BlockSpec((R, 1), lambda i: (i, 0)),
            pl.BlockSpec((1, f), lambda i: (0, 0)),
            pl.BlockSpec((1, f), lambda i: (0, 0)),
        ],
        out_shape=jax.ShapeDtypeStruct((n, f), jnp.float32),
        out_specs=pl.BlockSpec((R, f), lambda i: (i, 0)),
    )(p, p, g_prev, dinv, b_prev.reshape(1, f), w2)


def _tc3(p, g2, dinv, b2, n, f):
    g = n // R

    def body(p0_ref, p1_ref, g_ref, dinv_ref, b_ref, out_ref):
        agg = (p0_ref[...][0, :, 0:1] + p1_ref[...][0, :, 0:1]
               + g_ref[...][:, 0:1])
        out_ref[...] = jax.nn.sigmoid(dinv_ref[...] * agg + b_ref[...]) + 1e-6

    return pl.pallas_call(
        body,
        grid=(g,),
        in_specs=[
            pl.BlockSpec((1, R, f), lambda i: (0, i, 0)),
            pl.BlockSpec((1, R, f), lambda i: (1, i, 0)),
            pl.BlockSpec((R, f), lambda i: (i, 0)),
            pl.BlockSpec((R, 1), lambda i: (i, 0)),
            pl.BlockSpec((1, 1), lambda i: (0, 0)),
        ],
        out_shape=jax.ShapeDtypeStruct((n, 1), jnp.float32),
        out_specs=pl.BlockSpec((R, 1), lambda i: (i, 0)),
    )(p, p, g2, dinv, b2.reshape(1, 1))


def kernel(x, edge_index, batch, W0, b0, W1, b1, W2, b2):
    n, f = x.shape
    e = edge_index.shape[1]
    epad = _cdiv(e, NW * C) * NW * C
    ept = epad // NW
    acc_r = _cdiv(n + 1, NS * C) * NS * C

    src = jnp.concatenate(
        [edge_index[0].astype(jnp.int32), jnp.zeros((epad - e,), jnp.int32)])
    dst = jnp.concatenate(
        [edge_index[1].astype(jnp.int32), jnp.full((epad - e,), n, jnp.int32)])
    zeros_f = jnp.zeros((C, f), jnp.float32)
    ones_f = jnp.ones((C, f), jnp.float32)

    agg_f = _make_agg(f, ept, acc_r)
    deg_f = _make_deg(f, ept, acc_r)

    degp = deg_f(dst, zeros_f, ones_f)
    g0, dinv = _tc0(x, W0, degp, n, f)
    p0 = agg_f(g0, src, dst, zeros_f)
    g1 = _tc_mid(p0, g0, dinv, b0, W1, n, f)
    p1 = agg_f(g1, src, dst, zeros_f)
    g2 = _tc2(p1, g1, dinv, b1, W2, n, f)
    p2 = agg_f(g2, src, dst, zeros_f)
    return _tc3(p2, g2, dinv, b2, n, f)
